# baseline (speedup 1.0000x reference)
import jax
import jax.numpy as jnp
from jax import lax
from jax.experimental import pallas as pl
from jax.experimental.pallas import tpu as pltpu
from jax.experimental.pallas import tpu_sc as plsc

VOCAB = 100000
EMBED_DIM = 64
BATCH = 4096
HIST = 50
NUM_IDS = BATCH * HIST

NUM_WORKERS = 32
PER_WORKER = NUM_IDS // NUM_WORKERS
BATCHES_PER_WORKER = PER_WORKER // HIST
CB = 16
CHUNK = CB * HIST
NUM_CHUNKS = PER_WORKER // CHUNK
PAD_DIM = 128


def _sc_gather(table_padded, flat_ids):
    mesh = plsc.VectorSubcoreMesh(core_axis_name="c", subcore_axis_name="s")

    @pl.kernel(
        out_type=jax.ShapeDtypeStruct((BATCH, HIST, PAD_DIM), jnp.float32),
        mesh=mesh,
        scratch_types=[
            pltpu.VMEM((CHUNK,), jnp.int32),
            pltpu.VMEM((CHUNK,), jnp.int32),
            pltpu.VMEM((CHUNK, PAD_DIM), jnp.float32),
            pltpu.SemaphoreType.DMA,
            pltpu.SemaphoreType.DMA,
            pltpu.SemaphoreType.DMA,
            pltpu.SemaphoreType.DMA,
        ],
    )
    def k(table_hbm, ids_hbm, out_hbm,
          idx0, idx1, rows0,
          isem0, isem1, gsem0, ssem0):
        wid = lax.axis_index("s") * 2 + lax.axis_index("c")
        base = wid * PER_WORKER
        bbase = wid * BATCHES_PER_WORKER

        idx_bufs = [idx0, idx1]
        isems = [isem0, isem1]

        def idx_load(c, buf, sem):
            return pltpu.async_copy(
                ids_hbm.at[pl.ds(base + c * CHUNK, CHUNK)], buf, sem)

        def stores(c, sem):
            hs = []
            for b in range(CB):
                hs.append(pltpu.async_copy(
                    rows0.at[pl.ds(b * HIST, HIST)],
                    out_hbm.at[bbase + c * CB + b],
                    sem))
            return hs

        ih = [None, None]
        ih[0] = idx_load(0, idx_bufs[0], isems[0])
        if NUM_CHUNKS > 1:
            ih[1] = idx_load(1, idx_bufs[1], isems[1])

        store_hs = []
        for c in range(NUM_CHUNKS):
            cur = c & 1
            ih[cur].wait()
            for h in store_hs:
                h.wait()
            gh = pltpu.async_copy(table_hbm.at[idx_bufs[cur]], rows0, gsem0)
            if c + 2 < NUM_CHUNKS:
                ih[cur] = idx_load(c + 2, idx_bufs[cur], isems[cur])
            gh.wait()
            store_hs = stores(c, ssem0)

        for h in store_hs:
            h.wait()

    return k(table_padded, flat_ids)


def kernel(vocab_ids, table):
    flat_ids = vocab_ids.astype(jnp.int32).reshape(NUM_IDS)
    table_padded = jnp.pad(table, ((0, 0), (0, PAD_DIM - EMBED_DIM)))
    out = _sc_gather(table_padded, flat_ids)
    return out[:, :, :EMBED_DIM]


# speedup vs baseline: 1.0060x; 1.0060x over previous
import jax
import jax.numpy as jnp
from jax import lax
from jax.experimental import pallas as pl
from jax.experimental.pallas import tpu as pltpu
from jax.experimental.pallas import tpu_sc as plsc

VOCAB = 100000
EMBED_DIM = 64
BATCH = 4096
HIST = 50
NUM_IDS = BATCH * HIST

NUM_WORKERS = 32
PER_WORKER = NUM_IDS // NUM_WORKERS
BATCHES_PER_WORKER = PER_WORKER // HIST
CB = 8
CHUNK = CB * HIST
NUM_CHUNKS = PER_WORKER // CHUNK
PAD_DIM = 128


def _sc_gather(table_padded, flat_ids):
    mesh = plsc.VectorSubcoreMesh(core_axis_name="c", subcore_axis_name="s")

    @pl.kernel(
        out_type=jax.ShapeDtypeStruct((BATCH, HIST, PAD_DIM), jnp.float32),
        mesh=mesh,
        scratch_types=[
            pltpu.VMEM((CHUNK,), jnp.int32),
            pltpu.VMEM((CHUNK,), jnp.int32),
            pltpu.VMEM((CHUNK, PAD_DIM), jnp.float32),
            pltpu.VMEM((CHUNK, PAD_DIM), jnp.float32),
            pltpu.SemaphoreType.DMA,
            pltpu.SemaphoreType.DMA,
            pltpu.SemaphoreType.DMA,
            pltpu.SemaphoreType.DMA,
            pltpu.SemaphoreType.DMA,
            pltpu.SemaphoreType.DMA,
        ],
    )
    def k(table_hbm, ids_hbm, out_hbm,
          idx0, idx1, rows0, rows1,
          isem0, isem1, gsem0, gsem1, ssem0, ssem1):
        wid = lax.axis_index("s") * 2 + lax.axis_index("c")
        base = wid * PER_WORKER
        bbase = wid * BATCHES_PER_WORKER

        idx_bufs = [idx0, idx1]
        row_bufs = [rows0, rows1]
        isems = [isem0, isem1]
        gsems = [gsem0, gsem1]
        ssems = [ssem0, ssem1]

        def idx_load(c, buf, sem):
            return pltpu.async_copy(
                ids_hbm.at[pl.ds(base + c * CHUNK, CHUNK)], buf, sem)

        def gather(idx_buf, row_buf, sem):
            return pltpu.async_copy(table_hbm.at[idx_buf], row_buf, sem)

        def stores(c, row_buf, sem):
            hs = []
            for b in range(CB):
                hs.append(pltpu.async_copy(
                    row_buf.at[pl.ds(b * HIST, HIST)],
                    out_hbm.at[bbase + c * CB + b],
                    sem))
            return hs

        ih = [None, None]
        gh = [None, None]
        store_hs = [[], []]

        ih[0] = idx_load(0, idx_bufs[0], isems[0])
        ih[0].wait()
        gh[0] = gather(idx_bufs[0], row_bufs[0], gsems[0])
        if NUM_CHUNKS > 1:
            ih[1] = idx_load(1, idx_bufs[1], isems[1])

        for c in range(NUM_CHUNKS):
            cur = c & 1
            nxt = cur ^ 1
            gh[cur].wait()
            if c + 1 < NUM_CHUNKS:
                ih[nxt].wait()
                for h in store_hs[nxt]:
                    h.wait()
                store_hs[nxt] = []
                gh[nxt] = gather(idx_bufs[nxt], row_bufs[nxt], gsems[nxt])
                if c + 2 < NUM_CHUNKS:
                    ih[cur] = idx_load(c + 2, idx_bufs[cur], isems[cur])
            store_hs[cur] = stores(c, row_bufs[cur], ssems[cur])

        for hs in store_hs:
            for h in hs:
                h.wait()

    return k(table_padded, flat_ids)


def kernel(vocab_ids, table):
    flat_ids = vocab_ids.astype(jnp.int32).reshape(NUM_IDS)
    table_padded = jnp.pad(table, ((0, 0), (0, PAD_DIM - EMBED_DIM)))
    out = _sc_gather(table_padded, flat_ids)
    return out[:, :, :EMBED_DIM]
